# baseline (speedup 1.0000x reference)
import jax
import jax.numpy as jnp
from jax.experimental import pallas as pl
from jax.experimental.pallas import tpu as pltpu

_N = 3000
_NPAD = 3072
_BI = 256
_NBLK = _NPAD // _BI
_H = 16
_CUTS = 8
_GROUPS = 4
_EPS = 1e-5
_RADIUS = 9.0
_SENT = 8192.0
_INV_SQRT2 = 0.7071067811865476


def _lrelu(v):
    return jnp.where(v >= 0, v, 0.2 * v)


def _group_norm(z, gamma, beta):
    cnt = float(_N * (_H // _GROUPS))
    cols = _H // _GROUPS
    parts = []
    for g in range(_GROUPS):
        sl = z[:, g * cols:(g + 1) * cols]
        m = jnp.sum(sl) / cnt
        v = jnp.sum((sl - m) ** 2) / cnt
        parts.append((sl - m) * jax.lax.rsqrt(v + _EPS))
    return jnp.concatenate(parts, axis=1) * gamma + beta


def _fused_kernel(xyz_ref, nrm_ref, feat_ref, tall_ref,
                  wq1t_ref, bq1_ref, wq2t_ref, bq2_ref,
                  win1t_ref, bin1_ref, win2t_ref, bin2_ref, gin_ref, bein_ref,
                  a2t_ref, b2_ref,
                  wout1t_ref, bout1_ref, wout2t_ref, bout2_ref,
                  gout_ref, beout_ref,
                  wl1t_ref, bl1_ref, wl2t_ref, bl2_ref, wtt_ref, bt_ref,
                  out_ref, rjn_s, rj8t_s, p48w_s, ff_s, xi_s):
    i = pl.program_id(0)

    @pl.when(i == 0)
    def _prologue():
        p = xyz_ref[...] * (1.0 / _RADIUS)
        rjn_s[0:_N, 0:3] = p
        rjn_s[0:_N, 3:4] = jnp.ones((_N, 1), jnp.float32)
        rjn_s[0:_N, 4:5] = jnp.sum(p * p, axis=1, keepdims=True)
        rjn_s[0:_N, 5:8] = nrm_ref[...]
        fcol = lambda v: jnp.full((_NPAD - _N, 1), v, jnp.float32)
        rjn_s[_N:_NPAD, :] = jnp.concatenate(
            [fcol(_SENT), fcol(_SENT), fcol(_SENT), fcol(1.0),
             fcol(3.0 * _SENT * _SENT), fcol(0.0), fcol(0.0), fcol(0.0)],
            axis=1)
        rj8t_s[...] = jnp.transpose(rjn_s[...]).astype(jnp.bfloat16)

        feat = feat_ref[...]
        h1 = _lrelu(jnp.dot(feat, wq1t_ref[...],
                            preferred_element_type=jnp.float32) + bq1_ref[...])
        w = jnp.dot(h1, wq2t_ref[...],
                    preferred_element_type=jnp.float32) + bq2_ref[...]
        p48w_s[0:_N, 0:4] = rjn_s[0:_N, 0:4] * w
        p48w_s[0:_N, 4:8] = jnp.zeros((_N, 4), jnp.float32)
        p48w_s[_N:_NPAD, :] = jnp.zeros((_NPAD - _N, 8), jnp.float32)

        z = _lrelu(jnp.dot(feat, win1t_ref[...],
                           preferred_element_type=jnp.float32) + bin1_ref[...])
        z = _lrelu(jnp.dot(z, win2t_ref[...],
                           preferred_element_type=jnp.float32) + bin2_ref[...])
        f = _group_norm(z, gin_ref[...], bein_ref[...])
        zpad = jnp.zeros((_NPAD - _N, _H), jnp.bfloat16)
        for k in range(_CUTS):
            ff_s[k * _NPAD:k * _NPAD + _N, :] = (
                f * a2t_ref[k:k + 1, :]).astype(jnp.bfloat16)
            ff_s[k * _NPAD + _N:(k + 1) * _NPAD, :] = zpad
        ff_s[_CUTS * _NPAD:_CUTS * _NPAD + _N, :] = (
            f * b2_ref[...]).astype(jnp.bfloat16)
        ff_s[_CUTS * _NPAD + _N:(_CUTS + 1) * _NPAD, :] = zpad

    @pl.when((i >= 1) & (i <= _NBLK))
    def _main():
        off = (i - 1) * _BI
        blk = rjn_s[pl.ds(off, _BI), :]
        pxi, pyi, pzi = blk[:, 0:1], blk[:, 1:2], blk[:, 2:3]
        pp = blk[:, 4:5]
        nxi, nyi, nzi = blk[:, 5:6], blk[:, 6:7], blk[:, 7:8]

        zero = jnp.zeros_like(pxi)
        half = jnp.full_like(pxi, -0.5)
        two = jnp.full_like(pxi, 2.0)
        lhs_a = jnp.concatenate(
            [pxi, pyi, pzi, -0.5 * pp, half, zero, zero, zero], axis=1)
        lhs_b = jnp.concatenate(
            [zero, zero, zero, two, zero, -nxi, -nyi, -nzi], axis=1)
        lhs = jnp.concatenate([lhs_a, lhs_b], axis=0).astype(jnp.bfloat16)
        zq = jnp.dot(lhs, rj8t_s[...], preferred_element_type=jnp.float32)
        z = zq[:_BI, :]
        q = zq[_BI:, :]
        e = jnp.exp(z * q * q)

        s4 = jnp.dot(e * e, p48w_s[...], preferred_element_type=jnp.float32)
        ss = s4[:, 3:4]
        gx = s4[:, 0:1] - ss * pxi
        gy = s4[:, 1:2] - ss * pyi
        gz = s4[:, 2:3] - ss * pzi

        s = 2.0 * (nzi >= 0).astype(jnp.float32) - 1.0
        a = -1.0 / (s + nzi)
        b = nxi * nyi * a
        u0 = 1.0 + s * nxi * nxi * a
        u1 = s * b
        u2 = -s * nxi
        v0 = b
        v1 = s + nyi * nyi * a
        v2 = -nyi

        ex = u0 * gx + u1 * gy + u2 * gz + 1e-5
        ey = v0 * gx + v1 * gy + v2 * gz + 1e-5
        inv = 1.0 / jnp.maximum(jnp.sqrt(ex * ex + ey * ey), 1e-12)
        ex = ex * inv
        ey = ey * inv

        c = _INV_SQRT2
        b00, b01, b02 = nxi * c, nyi * c, nzi * c
        b10 = (ex * u0 + ey * v0) * c
        b11 = (ex * u1 + ey * v1) * c
        b12 = (ex * u2 + ey * v2) * c
        b20 = (ex * v0 - ey * u0) * c
        b21 = (ex * v1 - ey * u1) * c
        b22 = (ex * v2 - ey * u2) * c

        e0 = b00 * pxi + b01 * pyi + b02 * pzi
        e1 = b10 * pxi + b11 * pyi + b12 * pzi
        e2 = b20 * pxi + b21 * pyi + b22 * pzi

        ones = jnp.ones_like(pxi)
        basism = jnp.concatenate(
            [b00, b01, b02, b10, b11, b12, b20, b21, b22,
             -e0, -e1, -e2, ones], axis=1)
        lhs_all = jnp.dot(
            basism, tall_ref[...],
            preferred_element_type=jnp.float32).astype(jnp.bfloat16)

        rj4 = rj8t_s[0:4, :]
        acc = jnp.dot(e.astype(jnp.bfloat16),
                      ff_s[_CUTS * _NPAD:(_CUTS + 1) * _NPAD, :],
                      preferred_element_type=jnp.float32)
        for k in range(_CUTS):
            rk = jnp.dot(lhs_all[:, 4 * k:4 * (k + 1)], rj4,
                         preferred_element_type=jnp.float32)
            g = (jnp.maximum(rk, 0.0) * e).astype(jnp.bfloat16)
            acc = acc + jnp.dot(g, ff_s[k * _NPAD:(k + 1) * _NPAD, :],
                                preferred_element_type=jnp.float32)
        xi_s[pl.ds(off, _BI), :] = acc

    @pl.when(i == _NBLK + 1)
    def _epilogue():
        t = _lrelu(jnp.dot(xi_s[0:_N, :], wout1t_ref[...],
                           preferred_element_type=jnp.float32)
                   + bout1_ref[...])
        t = _lrelu(jnp.dot(t, wout2t_ref[...],
                           preferred_element_type=jnp.float32)
                   + bout2_ref[...])
        t = _group_norm(t, gout_ref[...], beout_ref[...])
        t = jnp.dot(jnp.maximum(jnp.dot(t, wl1t_ref[...],
                                        preferred_element_type=jnp.float32)
                                + bl1_ref[...], 0.0),
                    wl2t_ref[...],
                    preferred_element_type=jnp.float32) + bl2_ref[...]
        out_ref[...] = t + jnp.dot(feat_ref[...], wtt_ref[...],
                                   preferred_element_type=jnp.float32
                                   ) + bt_ref[...]


def kernel(surface_xyz, surface_normals, features, Wq1, bq1, Wq2, bq2,
           Win1, bin1, Win2, bin2, g_in, be_in, A1, B1, A2, B2,
           Wout1, bout1, Wout2, bout2, g_out, be_out,
           Wl1, bl1, Wl2, bl2, Wt, bt):
    row1 = lambda v: v.reshape(1, -1)

    eye3 = jnp.eye(3, dtype=jnp.float32)
    tcols = []
    for k in range(_CUTS):
        top = jnp.concatenate(
            [jnp.kron(A1[k].reshape(3, 1), eye3),
             jnp.zeros((9, 1), jnp.float32)], axis=1)
        mid = jnp.concatenate(
            [jnp.zeros((3, 3), jnp.float32), A1[k].reshape(3, 1)], axis=1)
        bot = jnp.concatenate(
            [jnp.zeros((1, 3), jnp.float32), B1[k].reshape(1, 1)], axis=1)
        tcols.append(jnp.concatenate([top, mid, bot], axis=0))
    tall = jnp.concatenate(tcols, axis=1)

    const = lambda i: (0, 0)
    full = lambda shp: pl.BlockSpec(shp, const)
    return pl.pallas_call(
        _fused_kernel,
        grid=(_NBLK + 2,),
        in_specs=[
            full((_N, 3)),
            full((_N, 3)),
            full((_N, _H)),
            full((13, 32)),
            full((_H, _H)), full((1, _H)), full((_H, 1)), full((1, 1)),
            full((_H, _H)), full((1, _H)), full((_H, _H)), full((1, _H)),
            full((1, _H)), full((1, _H)),
            full((_CUTS, _H)), full((1, _H)),
            full((_H, _H)), full((1, _H)), full((_H, _H)), full((1, _H)),
            full((1, _H)), full((1, _H)),
            full((_H, _H)), full((1, _H)), full((_H, _H)), full((1, _H)),
            full((_H, _H)), full((1, _H)),
        ],
        out_specs=pl.BlockSpec((_N, _H), const),
        out_shape=jax.ShapeDtypeStruct((_N, _H), jnp.float32),
        scratch_shapes=[
            pltpu.VMEM((_NPAD, 8), jnp.float32),
            pltpu.VMEM((8, _NPAD), jnp.bfloat16),
            pltpu.VMEM((_NPAD, 8), jnp.float32),
            pltpu.VMEM(((_CUTS + 1) * _NPAD, _H), jnp.bfloat16),
            pltpu.VMEM((_NPAD, _H), jnp.float32),
        ],
        compiler_params=pltpu.CompilerParams(
            dimension_semantics=("arbitrary",)),
    )(surface_xyz, surface_normals, features, tall,
      Wq1.T, row1(bq1), Wq2.T, bq2.reshape(1, 1),
      Win1.T, row1(bin1), Win2.T, row1(bin2), row1(g_in), row1(be_in),
      A2.T, row1(B2),
      Wout1.T, row1(bout1), Wout2.T, row1(bout2), row1(g_out), row1(be_out),
      Wl1.T, row1(bl1), Wl2.T, row1(bl2), Wt.T, row1(bt))


# speedup vs baseline: 1.0275x; 1.0275x over previous
import jax
import jax.numpy as jnp
from jax.experimental import pallas as pl
from jax.experimental.pallas import tpu as pltpu

_N = 3000
_NPAD = 3072
_BI = 512
_NBLK = _NPAD // _BI
_H = 16
_CUTS = 8
_GROUPS = 4
_EPS = 1e-5
_RADIUS = 9.0
_SENT = 8192.0
_INV_SQRT2 = 0.7071067811865476


def _lrelu(v):
    return jnp.where(v >= 0, v, 0.2 * v)


def _group_norm(z, gamma, beta):
    cnt = float(_N * (_H // _GROUPS))
    cols = _H // _GROUPS
    parts = []
    for g in range(_GROUPS):
        sl = z[:, g * cols:(g + 1) * cols]
        m = jnp.sum(sl) / cnt
        v = jnp.sum((sl - m) ** 2) / cnt
        parts.append((sl - m) * jax.lax.rsqrt(v + _EPS))
    return jnp.concatenate(parts, axis=1) * gamma + beta


def _fused_kernel(xyz_ref, nrm_ref, feat_ref, tall_ref,
                  wq1t_ref, bq1_ref, wq2t_ref, bq2_ref,
                  win1t_ref, bin1_ref, win2t_ref, bin2_ref, gin_ref, bein_ref,
                  a2t_ref, b2_ref,
                  wout1t_ref, bout1_ref, wout2t_ref, bout2_ref,
                  gout_ref, beout_ref,
                  wl1t_ref, bl1_ref, wl2t_ref, bl2_ref, wtt_ref, bt_ref,
                  out_ref, rjn_s, rj8t_s, p48w_s, ff_s, xi_s):
    i = pl.program_id(0)

    @pl.when(i == 0)
    def _prologue():
        p = xyz_ref[...] * (1.0 / _RADIUS)
        rjn_s[0:_N, 0:3] = p
        rjn_s[0:_N, 3:4] = jnp.ones((_N, 1), jnp.float32)
        rjn_s[0:_N, 4:5] = jnp.sum(p * p, axis=1, keepdims=True)
        rjn_s[0:_N, 5:8] = nrm_ref[...]
        fcol = lambda v: jnp.full((_NPAD - _N, 1), v, jnp.float32)
        rjn_s[_N:_NPAD, :] = jnp.concatenate(
            [fcol(_SENT), fcol(_SENT), fcol(_SENT), fcol(1.0),
             fcol(3.0 * _SENT * _SENT), fcol(0.0), fcol(0.0), fcol(0.0)],
            axis=1)
        rj8t_s[...] = jnp.transpose(rjn_s[...]).astype(jnp.bfloat16)

        feat = feat_ref[...]
        h1 = _lrelu(jnp.dot(feat, wq1t_ref[...],
                            preferred_element_type=jnp.float32) + bq1_ref[...])
        w = jnp.dot(h1, wq2t_ref[...],
                    preferred_element_type=jnp.float32) + bq2_ref[...]
        p48w_s[0:_N, 0:4] = rjn_s[0:_N, 0:4] * w
        p48w_s[0:_N, 4:8] = jnp.zeros((_N, 4), jnp.float32)
        p48w_s[_N:_NPAD, :] = jnp.zeros((_NPAD - _N, 8), jnp.float32)

        z = _lrelu(jnp.dot(feat, win1t_ref[...],
                           preferred_element_type=jnp.float32) + bin1_ref[...])
        z = _lrelu(jnp.dot(z, win2t_ref[...],
                           preferred_element_type=jnp.float32) + bin2_ref[...])
        f = _group_norm(z, gin_ref[...], bein_ref[...])
        zpad = jnp.zeros((_NPAD - _N, _H), jnp.bfloat16)
        for k in range(_CUTS):
            ff_s[k * _NPAD:k * _NPAD + _N, :] = (
                f * a2t_ref[k:k + 1, :]).astype(jnp.bfloat16)
            ff_s[k * _NPAD + _N:(k + 1) * _NPAD, :] = zpad
        ff_s[_CUTS * _NPAD:_CUTS * _NPAD + _N, :] = (
            f * b2_ref[...]).astype(jnp.bfloat16)
        ff_s[_CUTS * _NPAD + _N:(_CUTS + 1) * _NPAD, :] = zpad

    @pl.when((i >= 1) & (i <= _NBLK))
    def _main():
        off = (i - 1) * _BI
        blk = rjn_s[pl.ds(off, _BI), :]
        pxi, pyi, pzi = blk[:, 0:1], blk[:, 1:2], blk[:, 2:3]
        pp = blk[:, 4:5]
        nxi, nyi, nzi = blk[:, 5:6], blk[:, 6:7], blk[:, 7:8]

        zero = jnp.zeros_like(pxi)
        half = jnp.full_like(pxi, -0.5)
        two = jnp.full_like(pxi, 2.0)
        lhs_a = jnp.concatenate(
            [pxi, pyi, pzi, -0.5 * pp, half, zero, zero, zero], axis=1)
        lhs_b = jnp.concatenate(
            [zero, zero, zero, two, zero, -nxi, -nyi, -nzi], axis=1)
        lhs = jnp.concatenate([lhs_a, lhs_b], axis=0).astype(jnp.bfloat16)
        zq = jnp.dot(lhs, rj8t_s[...], preferred_element_type=jnp.float32)
        z = zq[:_BI, :]
        q = zq[_BI:, :]
        e = jnp.exp(z * q * q)

        s4 = jnp.dot((e * e).astype(jnp.bfloat16), p48w_s[...],
                     preferred_element_type=jnp.float32)
        ss = s4[:, 3:4]
        gx = s4[:, 0:1] - ss * pxi
        gy = s4[:, 1:2] - ss * pyi
        gz = s4[:, 2:3] - ss * pzi

        s = 2.0 * (nzi >= 0).astype(jnp.float32) - 1.0
        a = -1.0 / (s + nzi)
        b = nxi * nyi * a
        u0 = 1.0 + s * nxi * nxi * a
        u1 = s * b
        u2 = -s * nxi
        v0 = b
        v1 = s + nyi * nyi * a
        v2 = -nyi

        ex = u0 * gx + u1 * gy + u2 * gz + 1e-5
        ey = v0 * gx + v1 * gy + v2 * gz + 1e-5
        inv = 1.0 / jnp.maximum(jnp.sqrt(ex * ex + ey * ey), 1e-12)
        ex = ex * inv
        ey = ey * inv

        c = _INV_SQRT2
        b00, b01, b02 = nxi * c, nyi * c, nzi * c
        b10 = (ex * u0 + ey * v0) * c
        b11 = (ex * u1 + ey * v1) * c
        b12 = (ex * u2 + ey * v2) * c
        b20 = (ex * v0 - ey * u0) * c
        b21 = (ex * v1 - ey * u1) * c
        b22 = (ex * v2 - ey * u2) * c

        e0 = b00 * pxi + b01 * pyi + b02 * pzi
        e1 = b10 * pxi + b11 * pyi + b12 * pzi
        e2 = b20 * pxi + b21 * pyi + b22 * pzi

        ones = jnp.ones_like(pxi)
        basism = jnp.concatenate(
            [b00, b01, b02, b10, b11, b12, b20, b21, b22,
             -e0, -e1, -e2, ones], axis=1)
        lhs_all = jnp.dot(
            basism, tall_ref[...],
            preferred_element_type=jnp.float32).astype(jnp.bfloat16)

        rj4 = rj8t_s[0:4, :]
        acc = jnp.dot(e.astype(jnp.bfloat16),
                      ff_s[_CUTS * _NPAD:(_CUTS + 1) * _NPAD, :],
                      preferred_element_type=jnp.float32)
        for k in range(_CUTS):
            rk = jnp.dot(lhs_all[:, 4 * k:4 * (k + 1)], rj4,
                         preferred_element_type=jnp.float32)
            g = (jnp.maximum(rk, 0.0) * e).astype(jnp.bfloat16)
            acc = acc + jnp.dot(g, ff_s[k * _NPAD:(k + 1) * _NPAD, :],
                                preferred_element_type=jnp.float32)
        xi_s[pl.ds(off, _BI), :] = acc

    @pl.when(i == _NBLK + 1)
    def _epilogue():
        t = _lrelu(jnp.dot(xi_s[0:_N, :], wout1t_ref[...],
                           preferred_element_type=jnp.float32)
                   + bout1_ref[...])
        t = _lrelu(jnp.dot(t, wout2t_ref[...],
                           preferred_element_type=jnp.float32)
                   + bout2_ref[...])
        t = _group_norm(t, gout_ref[...], beout_ref[...])
        t = jnp.dot(jnp.maximum(jnp.dot(t, wl1t_ref[...],
                                        preferred_element_type=jnp.float32)
                                + bl1_ref[...], 0.0),
                    wl2t_ref[...],
                    preferred_element_type=jnp.float32) + bl2_ref[...]
        out_ref[...] = t + jnp.dot(feat_ref[...], wtt_ref[...],
                                   preferred_element_type=jnp.float32
                                   ) + bt_ref[...]


def kernel(surface_xyz, surface_normals, features, Wq1, bq1, Wq2, bq2,
           Win1, bin1, Win2, bin2, g_in, be_in, A1, B1, A2, B2,
           Wout1, bout1, Wout2, bout2, g_out, be_out,
           Wl1, bl1, Wl2, bl2, Wt, bt):
    row1 = lambda v: v.reshape(1, -1)

    eye3 = jnp.eye(3, dtype=jnp.float32)
    tcols = []
    for k in range(_CUTS):
        top = jnp.concatenate(
            [jnp.kron(A1[k].reshape(3, 1), eye3),
             jnp.zeros((9, 1), jnp.float32)], axis=1)
        mid = jnp.concatenate(
            [jnp.zeros((3, 3), jnp.float32), A1[k].reshape(3, 1)], axis=1)
        bot = jnp.concatenate(
            [jnp.zeros((1, 3), jnp.float32), B1[k].reshape(1, 1)], axis=1)
        tcols.append(jnp.concatenate([top, mid, bot], axis=0))
    tall = jnp.concatenate(tcols, axis=1)

    const = lambda i: (0, 0)
    full = lambda shp: pl.BlockSpec(shp, const)
    return pl.pallas_call(
        _fused_kernel,
        grid=(_NBLK + 2,),
        in_specs=[
            full((_N, 3)),
            full((_N, 3)),
            full((_N, _H)),
            full((13, 32)),
            full((_H, _H)), full((1, _H)), full((_H, 1)), full((1, 1)),
            full((_H, _H)), full((1, _H)), full((_H, _H)), full((1, _H)),
            full((1, _H)), full((1, _H)),
            full((_CUTS, _H)), full((1, _H)),
            full((_H, _H)), full((1, _H)), full((_H, _H)), full((1, _H)),
            full((1, _H)), full((1, _H)),
            full((_H, _H)), full((1, _H)), full((_H, _H)), full((1, _H)),
            full((_H, _H)), full((1, _H)),
        ],
        out_specs=pl.BlockSpec((_N, _H), const),
        out_shape=jax.ShapeDtypeStruct((_N, _H), jnp.float32),
        scratch_shapes=[
            pltpu.VMEM((_NPAD, 8), jnp.float32),
            pltpu.VMEM((8, _NPAD), jnp.bfloat16),
            pltpu.VMEM((_NPAD, 8), jnp.float32),
            pltpu.VMEM(((_CUTS + 1) * _NPAD, _H), jnp.bfloat16),
            pltpu.VMEM((_NPAD, _H), jnp.float32),
        ],
        compiler_params=pltpu.CompilerParams(
            dimension_semantics=("arbitrary",)),
    )(surface_xyz, surface_normals, features, tall,
      Wq1.T, row1(bq1), Wq2.T, bq2.reshape(1, 1),
      Win1.T, row1(bin1), Win2.T, row1(bin2), row1(g_in), row1(be_in),
      A2.T, row1(B2),
      Wout1.T, row1(bout1), Wout2.T, row1(bout2), row1(g_out), row1(be_out),
      Wl1.T, row1(bl1), Wl2.T, row1(bl2), Wt.T, row1(bt))


# speedup vs baseline: 1.0655x; 1.0370x over previous
import jax
import jax.numpy as jnp
from jax.experimental import pallas as pl
from jax.experimental.pallas import tpu as pltpu

_N = 3000
_NPAD = 3072
_BI = 512
_NBLK = _NPAD // _BI
_H = 16
_CUTS = 8
_GROUPS = 4
_EPS = 1e-5
_RADIUS = 9.0
_SENT = 8192.0
_INV_SQRT2 = 0.7071067811865476


def _lrelu(v):
    return jnp.where(v >= 0, v, 0.2 * v)


def _group_norm(z, gamma, beta):
    cnt = float(_N * (_H // _GROUPS))
    ones_row = jnp.ones((1, _N), jnp.float32)
    ri = jax.lax.broadcasted_iota(jnp.int32, (_H, _H), 0)
    ci = jax.lax.broadcasted_iota(jnp.int32, (_H, _H), 1)
    mix = ((ri // (_H // _GROUPS)) == (ci // (_H // _GROUPS))
           ).astype(jnp.float32)
    csum = jnp.dot(ones_row, z, preferred_element_type=jnp.float32)
    m = jnp.dot(csum, mix, preferred_element_type=jnp.float32) / cnt
    t = z - m
    csq = jnp.dot(ones_row, t * t, preferred_element_type=jnp.float32)
    v = jnp.dot(csq, mix, preferred_element_type=jnp.float32) / cnt
    return t * jax.lax.rsqrt(v + _EPS) * gamma + beta


def _fused_kernel(xyz_ref, nrm_ref, feat_ref, tall_ref,
                  wq1t_ref, bq1_ref, wq2t_ref, bq2_ref,
                  win1t_ref, bin1_ref, win2t_ref, bin2_ref, gin_ref, bein_ref,
                  a2t_ref, b2_ref,
                  wout1t_ref, bout1_ref, wout2t_ref, bout2_ref,
                  gout_ref, beout_ref,
                  wl1t_ref, bl1_ref, wl2t_ref, bl2_ref, wtt_ref, bt_ref,
                  out_ref, rjn_s, rj8t_s, p48w_s, ff_s, xi_s):
    i = pl.program_id(0)

    @pl.when(i == 0)
    def _prologue():
        p = xyz_ref[...] * (1.0 / _RADIUS)
        rjn_s[0:_N, 0:3] = p
        rjn_s[0:_N, 3:4] = jnp.ones((_N, 1), jnp.float32)
        rjn_s[0:_N, 4:5] = jnp.sum(p * p, axis=1, keepdims=True)
        rjn_s[0:_N, 5:8] = nrm_ref[...]
        fcol = lambda v: jnp.full((_NPAD - _N, 1), v, jnp.float32)
        rjn_s[_N:_NPAD, :] = jnp.concatenate(
            [fcol(_SENT), fcol(_SENT), fcol(_SENT), fcol(1.0),
             fcol(3.0 * _SENT * _SENT), fcol(0.0), fcol(0.0), fcol(0.0)],
            axis=1)
        rj8t_s[...] = jnp.transpose(rjn_s[...]).astype(jnp.bfloat16)

        feat = feat_ref[...]
        h1 = _lrelu(jnp.dot(feat, wq1t_ref[...],
                            preferred_element_type=jnp.float32) + bq1_ref[...])
        w = jnp.dot(h1, wq2t_ref[...],
                    preferred_element_type=jnp.float32) + bq2_ref[...]
        p48w_s[0:_N, 0:4] = rjn_s[0:_N, 0:4] * w
        p48w_s[0:_N, 4:8] = jnp.zeros((_N, 4), jnp.float32)
        p48w_s[_N:_NPAD, :] = jnp.zeros((_NPAD - _N, 8), jnp.float32)

        z = _lrelu(jnp.dot(feat, win1t_ref[...],
                           preferred_element_type=jnp.float32) + bin1_ref[...])
        z = _lrelu(jnp.dot(z, win2t_ref[...],
                           preferred_element_type=jnp.float32) + bin2_ref[...])
        f = _group_norm(z, gin_ref[...], bein_ref[...])
        zpad = jnp.zeros((_NPAD - _N, _H), jnp.bfloat16)
        for k in range(_CUTS):
            ff_s[k * _NPAD:k * _NPAD + _N, :] = (
                f * a2t_ref[k:k + 1, :]).astype(jnp.bfloat16)
            ff_s[k * _NPAD + _N:(k + 1) * _NPAD, :] = zpad
        ff_s[_CUTS * _NPAD:_CUTS * _NPAD + _N, :] = (
            f * b2_ref[...]).astype(jnp.bfloat16)
        ff_s[_CUTS * _NPAD + _N:(_CUTS + 1) * _NPAD, :] = zpad

    @pl.when((i >= 1) & (i <= _NBLK))
    def _main():
        off = (i - 1) * _BI
        blk = rjn_s[pl.ds(off, _BI), :]
        pxi, pyi, pzi = blk[:, 0:1], blk[:, 1:2], blk[:, 2:3]
        pp = blk[:, 4:5]
        nxi, nyi, nzi = blk[:, 5:6], blk[:, 6:7], blk[:, 7:8]

        zero = jnp.zeros_like(pxi)
        half = jnp.full_like(pxi, -0.5)
        two = jnp.full_like(pxi, 2.0)
        lhs_a = jnp.concatenate(
            [pxi, pyi, pzi, -0.5 * pp, half, zero, zero, zero], axis=1)
        lhs_b = jnp.concatenate(
            [zero, zero, zero, two, zero, -nxi, -nyi, -nzi], axis=1)
        lhs = jnp.concatenate([lhs_a, lhs_b], axis=0).astype(jnp.bfloat16)
        zq = jnp.dot(lhs, rj8t_s[...], preferred_element_type=jnp.float32)
        z = zq[:_BI, :]
        q = zq[_BI:, :]
        e = jnp.exp(z * q * q)

        s4 = jnp.dot((e * e).astype(jnp.bfloat16), p48w_s[...],
                     preferred_element_type=jnp.float32)
        ss = s4[:, 3:4]
        gx = s4[:, 0:1] - ss * pxi
        gy = s4[:, 1:2] - ss * pyi
        gz = s4[:, 2:3] - ss * pzi

        s = 2.0 * (nzi >= 0).astype(jnp.float32) - 1.0
        a = -1.0 / (s + nzi)
        b = nxi * nyi * a
        u0 = 1.0 + s * nxi * nxi * a
        u1 = s * b
        u2 = -s * nxi
        v0 = b
        v1 = s + nyi * nyi * a
        v2 = -nyi

        ex = u0 * gx + u1 * gy + u2 * gz + 1e-5
        ey = v0 * gx + v1 * gy + v2 * gz + 1e-5
        inv = 1.0 / jnp.maximum(jnp.sqrt(ex * ex + ey * ey), 1e-12)
        ex = ex * inv
        ey = ey * inv

        c = _INV_SQRT2
        b00, b01, b02 = nxi * c, nyi * c, nzi * c
        b10 = (ex * u0 + ey * v0) * c
        b11 = (ex * u1 + ey * v1) * c
        b12 = (ex * u2 + ey * v2) * c
        b20 = (ex * v0 - ey * u0) * c
        b21 = (ex * v1 - ey * u1) * c
        b22 = (ex * v2 - ey * u2) * c

        e0 = b00 * pxi + b01 * pyi + b02 * pzi
        e1 = b10 * pxi + b11 * pyi + b12 * pzi
        e2 = b20 * pxi + b21 * pyi + b22 * pzi

        ones = jnp.ones_like(pxi)
        basism = jnp.concatenate(
            [b00, b01, b02, b10, b11, b12, b20, b21, b22,
             -e0, -e1, -e2, ones], axis=1)
        lhs_all = jnp.dot(
            basism, tall_ref[...],
            preferred_element_type=jnp.float32).astype(jnp.bfloat16)

        rj4 = rj8t_s[0:4, :]
        e16 = e.astype(jnp.bfloat16)
        zero16 = jnp.zeros((), jnp.bfloat16)
        acc = jnp.dot(e16, ff_s[_CUTS * _NPAD:(_CUTS + 1) * _NPAD, :],
                      preferred_element_type=jnp.float32)
        for k in range(_CUTS):
            rk = jnp.dot(lhs_all[:, 4 * k:4 * (k + 1)], rj4,
                         preferred_element_type=jnp.float32)
            g = jnp.maximum(rk.astype(jnp.bfloat16), zero16) * e16
            acc = acc + jnp.dot(g, ff_s[k * _NPAD:(k + 1) * _NPAD, :],
                                preferred_element_type=jnp.float32)
        xi_s[pl.ds(off, _BI), :] = acc

    @pl.when(i == _NBLK + 1)
    def _epilogue():
        t = _lrelu(jnp.dot(xi_s[0:_N, :], wout1t_ref[...],
                           preferred_element_type=jnp.float32)
                   + bout1_ref[...])
        t = _lrelu(jnp.dot(t, wout2t_ref[...],
                           preferred_element_type=jnp.float32)
                   + bout2_ref[...])
        t = _group_norm(t, gout_ref[...], beout_ref[...])
        t = jnp.dot(jnp.maximum(jnp.dot(t, wl1t_ref[...],
                                        preferred_element_type=jnp.float32)
                                + bl1_ref[...], 0.0),
                    wl2t_ref[...],
                    preferred_element_type=jnp.float32) + bl2_ref[...]
        out_ref[...] = t + jnp.dot(feat_ref[...], wtt_ref[...],
                                   preferred_element_type=jnp.float32
                                   ) + bt_ref[...]


def kernel(surface_xyz, surface_normals, features, Wq1, bq1, Wq2, bq2,
           Win1, bin1, Win2, bin2, g_in, be_in, A1, B1, A2, B2,
           Wout1, bout1, Wout2, bout2, g_out, be_out,
           Wl1, bl1, Wl2, bl2, Wt, bt):
    row1 = lambda v: v.reshape(1, -1)

    eye3 = jnp.eye(3, dtype=jnp.float32)
    tcols = []
    for k in range(_CUTS):
        top = jnp.concatenate(
            [jnp.kron(A1[k].reshape(3, 1), eye3),
             jnp.zeros((9, 1), jnp.float32)], axis=1)
        mid = jnp.concatenate(
            [jnp.zeros((3, 3), jnp.float32), A1[k].reshape(3, 1)], axis=1)
        bot = jnp.concatenate(
            [jnp.zeros((1, 3), jnp.float32), B1[k].reshape(1, 1)], axis=1)
        tcols.append(jnp.concatenate([top, mid, bot], axis=0))
    tall = jnp.concatenate(tcols, axis=1)

    const = lambda i: (0, 0)
    full = lambda shp: pl.BlockSpec(shp, const)
    return pl.pallas_call(
        _fused_kernel,
        grid=(_NBLK + 2,),
        in_specs=[
            full((_N, 3)),
            full((_N, 3)),
            full((_N, _H)),
            full((13, 32)),
            full((_H, _H)), full((1, _H)), full((_H, 1)), full((1, 1)),
            full((_H, _H)), full((1, _H)), full((_H, _H)), full((1, _H)),
            full((1, _H)), full((1, _H)),
            full((_CUTS, _H)), full((1, _H)),
            full((_H, _H)), full((1, _H)), full((_H, _H)), full((1, _H)),
            full((1, _H)), full((1, _H)),
            full((_H, _H)), full((1, _H)), full((_H, _H)), full((1, _H)),
            full((_H, _H)), full((1, _H)),
        ],
        out_specs=pl.BlockSpec((_N, _H), const),
        out_shape=jax.ShapeDtypeStruct((_N, _H), jnp.float32),
        scratch_shapes=[
            pltpu.VMEM((_NPAD, 8), jnp.float32),
            pltpu.VMEM((8, _NPAD), jnp.bfloat16),
            pltpu.VMEM((_NPAD, 8), jnp.float32),
            pltpu.VMEM(((_CUTS + 1) * _NPAD, _H), jnp.bfloat16),
            pltpu.VMEM((_NPAD, _H), jnp.float32),
        ],
        compiler_params=pltpu.CompilerParams(
            dimension_semantics=("arbitrary",)),
    )(surface_xyz, surface_normals, features, tall,
      Wq1.T, row1(bq1), Wq2.T, bq2.reshape(1, 1),
      Win1.T, row1(bin1), Win2.T, row1(bin2), row1(g_in), row1(be_in),
      A2.T, row1(B2),
      Wout1.T, row1(bout1), Wout2.T, row1(bout2), row1(g_out), row1(be_out),
      Wl1.T, row1(bl1), Wl2.T, row1(bl2), Wt.T, row1(bt))
